# baseline (speedup 1.0000x reference)
import functools

import jax
import jax.numpy as jnp
from jax import lax
from jax.experimental import pallas as pl
from jax.experimental.pallas import tpu as pltpu
from jax.experimental.pallas import tpu_sc as plsc

N = 10000
D = 128
E = 320000

NC = 2
NS = 16
NW = NC * NS

C = 64
CHW = 160
EW = CHW * C
EP = EW * NW

ACC_ROWS = 10240
ROWS_PER_TILE = ACC_ROWS // NS
CP = 64
NCP = ROWS_PER_TILE // CP

NB = 3
HC = 16
NBATCH = CHW // HC

DEG_ACC = 16384
DEG_PER_TILE = DEG_ACC // NS
DC = 128
DCH = EW // DC

_mesh = plsc.VectorSubcoreMesh(
    core_axis_name="c", subcore_axis_name="s", num_cores=NC, num_subcores=NS
)


@functools.partial(
    pl.kernel,
    out_type=jax.ShapeDtypeStruct((NC * DEG_ACC,), jnp.float32),
    mesh=_mesh,
    scratch_types=[
        pltpu.VMEM((DCH, DC), jnp.int32),
        pltpu.VMEM((DC,), jnp.float32),
        pltpu.VMEM((DEG_PER_TILE,), jnp.float32),
        pltpu.VMEM_SHARED((DEG_ACC,), jnp.float32),
    ],
)
def _sc_degree(dst_hbm, ones_hbm, zeros_hbm, out_hbm, didx2, ones_v, dbuf, acc):
    c = lax.axis_index("c")
    s = lax.axis_index("s")
    w = c * NS + s
    pltpu.sync_copy(dst_hbm.at[pl.ds(w * DCH, DCH)], didx2)
    pltpu.sync_copy(zeros_hbm, dbuf)
    pltpu.sync_copy(dbuf, acc.at[pl.ds(s * DEG_PER_TILE, DEG_PER_TILE)])
    pltpu.sync_copy(ones_hbm, ones_v)
    plsc.subcore_barrier()

    def body(j, carry):
        pltpu.sync_copy(ones_v, acc.at[didx2.at[j]], add=True)
        return carry

    lax.fori_loop(0, DCH, body, 0)
    plsc.subcore_barrier()
    pltpu.sync_copy(acc.at[pl.ds(s * DEG_PER_TILE, DEG_PER_TILE)], dbuf)
    pltpu.sync_copy(dbuf, out_hbm.at[pl.ds(c * DEG_ACC + s * DEG_PER_TILE,
                                           DEG_PER_TILE)])


@functools.partial(
    pl.kernel,
    out_type=[
        jax.ShapeDtypeStruct((NC, ACC_ROWS, D), jnp.float32),
        jax.ShapeDtypeStruct((EP, D), jnp.float32),
    ],
    mesh=_mesh,
    scratch_types=[
        pltpu.VMEM((HC, C), jnp.int32),
        pltpu.VMEM((HC, C), jnp.int32),
        pltpu.VMEM((NB, C, D), jnp.float32),
        pltpu.VMEM_SHARED((ACC_ROWS, D), jnp.float32),
        pltpu.SemaphoreType.DMA,
        pltpu.SemaphoreType.DMA,
    ],
)
def _sc_scatter(g_hbm, src_hbm, dst_hbm, zrows_hbm, out_hbm, msg_hbm,
                sidx2, didx2, rows, spb, sem, sem2):
    c = lax.axis_index("c")
    s = lax.axis_index("s")
    w = c * NS + s
    for j in range(NCP):
        r0 = s * ROWS_PER_TILE + j * CP
        pltpu.sync_copy(g_hbm.at[pl.ds(r0, CP)], rows.at[0])
        pltpu.sync_copy(rows.at[0], spb.at[pl.ds(r0, CP)])
    plsc.subcore_barrier()

    def phase_a(h, carry):
        base = w * EW + h * HC * C
        pltpu.sync_copy(src_hbm.at[w, pl.ds(h * HC, HC)], sidx2)
        pltpu.async_copy(spb.at[sidx2.at[0]], rows.at[0], sem)
        for j in range(HC):
            b = j % NB
            if j >= 2:
                pltpu.make_async_copy(
                    rows.at[(j - 2) % NB],
                    msg_hbm.at[pl.ds(base + (j - 2) * C, C)], sem2).wait()
            if j + 1 < HC:
                pltpu.async_copy(spb.at[sidx2.at[j + 1]],
                                 rows.at[(j + 1) % NB], sem)
            pltpu.make_async_copy(spb.at[sidx2.at[j]], rows.at[b], sem).wait()
            pltpu.async_copy(rows.at[b],
                             msg_hbm.at[pl.ds(base + j * C, C)], sem2)
        for j in (HC - 2, HC - 1):
            pltpu.make_async_copy(
                rows.at[j % NB],
                msg_hbm.at[pl.ds(base + j * C, C)], sem2).wait()
        return carry

    lax.fori_loop(0, NBATCH, phase_a, 0)
    plsc.subcore_barrier()

    pltpu.sync_copy(zrows_hbm, rows.at[0])
    for j in range(NCP):
        pltpu.sync_copy(rows.at[0], spb.at[pl.ds(s * ROWS_PER_TILE + j * CP, CP)])
    plsc.subcore_barrier()

    def phase_b(h, carry):
        base = w * EW + h * HC * C
        pltpu.sync_copy(dst_hbm.at[w, pl.ds(h * HC, HC)], didx2)
        pltpu.async_copy(msg_hbm.at[pl.ds(base, C)], rows.at[0], sem)
        for j in range(HC):
            b = j % NB
            if j + 1 < HC:
                pltpu.async_copy(msg_hbm.at[pl.ds(base + (j + 1) * C, C)],
                                 rows.at[(j + 1) % NB], sem)
            pltpu.make_async_copy(msg_hbm.at[pl.ds(base + j * C, C)],
                                  rows.at[b], sem).wait()
            pltpu.sync_copy(rows.at[b], spb.at[didx2.at[j]], add=True)
        return carry

    lax.fori_loop(0, NBATCH, phase_b, 0)
    plsc.subcore_barrier()

    for j in range(NCP):
        r0 = s * ROWS_PER_TILE + j * CP
        pltpu.sync_copy(spb.at[pl.ds(r0, CP)], rows.at[0])
        pltpu.sync_copy(rows.at[0], out_hbm.at[c, pl.ds(r0, CP)])


BN = 400
GRID = N // BN


def _tc_scale_matmul_body(degp_ref, x_ref, w_ref, o_ref):
    dinv = lax.rsqrt(degp_ref[0] + degp_ref[1] + 1.0)
    o_ref[...] = dinv * jnp.dot(x_ref[...], w_ref[...],
                                preferred_element_type=jnp.float32)


def _tc_mid_body(degp_ref, p0_ref, p1_ref, g_ref, w_ref, b_ref, o_ref):
    dinv = lax.rsqrt(degp_ref[0] + degp_ref[1] + 1.0)
    h = dinv * (p0_ref[0] + p1_ref[0] + g_ref[...]) + b_ref[...]
    h = jnp.maximum(h, 0.0)
    o_ref[...] = dinv * jnp.dot(h, w_ref[...],
                                preferred_element_type=jnp.float32)


def _tc_final_body(degp_ref, p0_ref, p1_ref, g_ref, b_ref, o_ref):
    dinv = lax.rsqrt(degp_ref[0] + degp_ref[1] + 1.0)
    z = dinv * (p0_ref[0] + p1_ref[0] + g_ref[...]) + b_ref[...]
    m = jnp.max(z, axis=1, keepdims=True)
    e = jnp.exp(z - m)
    lse = jnp.log(jnp.sum(e, axis=1, keepdims=True)) + m
    o_ref[...] = z - lse


_deg_spec = pl.BlockSpec((2, BN, 1), lambda i: (0, i, 0))
_row_spec = pl.BlockSpec((BN, D), lambda i: (i, 0))
_part_spec0 = pl.BlockSpec((1, BN, D), lambda i: (0, i, 0))
_part_spec1 = pl.BlockSpec((1, BN, D), lambda i: (1, i, 0))
_w_spec = pl.BlockSpec((D, D), lambda i: (0, 0))
_b_spec = pl.BlockSpec((1, D), lambda i: (0, 0))
_out_f32 = jax.ShapeDtypeStruct((N, D), jnp.float32)


def _tc_scale_matmul(degp, x, w):
    return pl.pallas_call(
        _tc_scale_matmul_body,
        grid=(GRID,),
        in_specs=[_deg_spec, _row_spec, _w_spec],
        out_specs=_row_spec,
        out_shape=_out_f32,
    )(degp, x, w)


def _tc_mid(degp, part, g, w, b):
    return pl.pallas_call(
        _tc_mid_body,
        grid=(GRID,),
        in_specs=[_deg_spec, _part_spec0, _part_spec1, _row_spec, _w_spec,
                  _b_spec],
        out_specs=_row_spec,
        out_shape=_out_f32,
    )(degp, part, part, g, w, b)


def _tc_final(degp, part, g, b):
    return pl.pallas_call(
        _tc_final_body,
        grid=(GRID,),
        in_specs=[_deg_spec, _part_spec0, _part_spec1, _row_spec, _b_spec],
        out_specs=_row_spec,
        out_shape=_out_f32,
    )(degp, part, part, g, b)


def kernel(x, edge_index, W1, b1, W2, b2):
    x = x.astype(jnp.float32)
    src = edge_index[0].astype(jnp.int32)
    dst = edge_index[1].astype(jnp.int32)
    pad = EP - E
    srcp = jnp.concatenate([src, jnp.zeros((pad,), jnp.int32)])
    dstp = jnp.concatenate([dst, jnp.full((pad,), N, jnp.int32)])
    src3 = srcp.reshape(NW, CHW, C)
    dst3 = dstp.reshape(NW, CHW, C)
    dst_deg = dstp.reshape(NW * DCH, DC)

    ones_c = jnp.ones((DC,), jnp.float32)
    zeros_deg = jnp.zeros((DEG_PER_TILE,), jnp.float32)
    zeros_rows = jnp.zeros((CP, D), jnp.float32)
    rpad = jnp.zeros((ACC_ROWS - N, D), jnp.float32)

    degp = _sc_degree(dst_deg, ones_c, zeros_deg)
    degp = degp.reshape(NC, DEG_ACC)[:, :N].reshape(NC, N, 1)

    g1 = _tc_scale_matmul(degp, x, W1)
    g1p = jnp.concatenate([g1, rpad])
    part1, _ = _sc_scatter(g1p, src3, dst3, zeros_rows)
    g2 = _tc_mid(degp, part1, g1, W2, b1.reshape(1, D))
    g2p = jnp.concatenate([g2, rpad])
    part2, _ = _sc_scatter(g2p, src3, dst3, zeros_rows)
    return _tc_final(degp, part2, g2, b2.reshape(1, D))


# speedup vs baseline: 1.0331x; 1.0331x over previous
import functools

import jax
import jax.numpy as jnp
from jax import lax
from jax.experimental import pallas as pl
from jax.experimental.pallas import tpu as pltpu
from jax.experimental.pallas import tpu_sc as plsc

N = 10000
D = 128
E = 320000

NC = 2
NS = 16
NW = NC * NS

C = 64
CHW = 160
EW = CHW * C
EP = EW * NW

ACC_ROWS = 10240
ROWS_PER_TILE = ACC_ROWS // NS
CP = 64
NCP = ROWS_PER_TILE // CP

NB = 4
HC = 32
NBATCH = CHW // HC

DEG_ACC = 16384
DEG_PER_TILE = DEG_ACC // NS
DC = 128
DCH = EW // DC

_mesh = plsc.VectorSubcoreMesh(
    core_axis_name="c", subcore_axis_name="s", num_cores=NC, num_subcores=NS
)


@functools.partial(
    pl.kernel,
    out_type=jax.ShapeDtypeStruct((NC * DEG_ACC,), jnp.float32),
    mesh=_mesh,
    scratch_types=[
        pltpu.VMEM((DCH, DC), jnp.int32),
        pltpu.VMEM((DC,), jnp.float32),
        pltpu.VMEM((DEG_PER_TILE,), jnp.float32),
        pltpu.VMEM_SHARED((DEG_ACC,), jnp.float32),
    ],
)
def _sc_degree(dst_hbm, ones_hbm, zeros_hbm, out_hbm, didx2, ones_v, dbuf, acc):
    c = lax.axis_index("c")
    s = lax.axis_index("s")
    w = c * NS + s
    pltpu.sync_copy(dst_hbm.at[pl.ds(w * DCH, DCH)], didx2)
    pltpu.sync_copy(zeros_hbm, dbuf)
    pltpu.sync_copy(dbuf, acc.at[pl.ds(s * DEG_PER_TILE, DEG_PER_TILE)])
    pltpu.sync_copy(ones_hbm, ones_v)
    plsc.subcore_barrier()

    def body(j, carry):
        pltpu.sync_copy(ones_v, acc.at[didx2.at[j]], add=True)
        return carry

    lax.fori_loop(0, DCH, body, 0)
    plsc.subcore_barrier()
    pltpu.sync_copy(acc.at[pl.ds(s * DEG_PER_TILE, DEG_PER_TILE)], dbuf)
    pltpu.sync_copy(dbuf, out_hbm.at[pl.ds(c * DEG_ACC + s * DEG_PER_TILE,
                                           DEG_PER_TILE)])


@functools.partial(
    pl.kernel,
    out_type=[
        jax.ShapeDtypeStruct((NC, ACC_ROWS, D), jnp.float32),
        jax.ShapeDtypeStruct((EP, D), jnp.float32),
    ],
    mesh=_mesh,
    scratch_types=[
        pltpu.VMEM((HC, C), jnp.int32),
        pltpu.VMEM((HC, C), jnp.int32),
        pltpu.VMEM((NB, C, D), jnp.float32),
        pltpu.VMEM_SHARED((ACC_ROWS, D), jnp.float32),
        pltpu.SemaphoreType.DMA,
        pltpu.SemaphoreType.DMA,
    ],
)
def _sc_scatter(g_hbm, src_hbm, dst_hbm, zrows_hbm, out_hbm, msg_hbm,
                sidx2, didx2, rows, spb, sem, sem2):
    c = lax.axis_index("c")
    s = lax.axis_index("s")
    w = c * NS + s
    for j in range(NCP):
        r0 = s * ROWS_PER_TILE + j * CP
        pltpu.sync_copy(g_hbm.at[pl.ds(r0, CP)], rows.at[0])
        pltpu.sync_copy(rows.at[0], spb.at[pl.ds(r0, CP)])
    plsc.subcore_barrier()

    def phase_a(h, carry):
        base = w * EW + h * HC * C
        pltpu.sync_copy(src_hbm.at[w, pl.ds(h * HC, HC)], sidx2)
        pltpu.async_copy(spb.at[sidx2.at[0]], rows.at[0], sem)
        for j in range(HC):
            b = j % NB
            if j >= NB - 1:
                pltpu.make_async_copy(
                    rows.at[(j - NB + 1) % NB],
                    msg_hbm.at[pl.ds(base + (j - NB + 1) * C, C)], sem2).wait()
            if j + 1 < HC:
                pltpu.async_copy(spb.at[sidx2.at[j + 1]],
                                 rows.at[(j + 1) % NB], sem)
            pltpu.make_async_copy(spb.at[sidx2.at[j]], rows.at[b], sem).wait()
            pltpu.async_copy(rows.at[b],
                             msg_hbm.at[pl.ds(base + j * C, C)], sem2)
        for j in range(HC - NB + 1, HC):
            pltpu.make_async_copy(
                rows.at[j % NB],
                msg_hbm.at[pl.ds(base + j * C, C)], sem2).wait()
        return carry

    lax.fori_loop(0, NBATCH, phase_a, 0)
    plsc.subcore_barrier()

    pltpu.sync_copy(zrows_hbm, rows.at[0])
    for j in range(NCP):
        pltpu.sync_copy(rows.at[0], spb.at[pl.ds(s * ROWS_PER_TILE + j * CP, CP)])
    plsc.subcore_barrier()

    def phase_b(h, carry):
        base = w * EW + h * HC * C
        pltpu.sync_copy(dst_hbm.at[w, pl.ds(h * HC, HC)], didx2)
        pltpu.async_copy(msg_hbm.at[pl.ds(base, C)], rows.at[0], sem)
        for j in range(HC):
            b = j % NB
            if j + 1 < HC:
                pltpu.async_copy(msg_hbm.at[pl.ds(base + (j + 1) * C, C)],
                                 rows.at[(j + 1) % NB], sem)
            pltpu.make_async_copy(msg_hbm.at[pl.ds(base + j * C, C)],
                                  rows.at[b], sem).wait()
            pltpu.sync_copy(rows.at[b], spb.at[didx2.at[j]], add=True)
        return carry

    lax.fori_loop(0, NBATCH, phase_b, 0)
    plsc.subcore_barrier()

    for j in range(NCP):
        r0 = s * ROWS_PER_TILE + j * CP
        pltpu.sync_copy(spb.at[pl.ds(r0, CP)], rows.at[0])
        pltpu.sync_copy(rows.at[0], out_hbm.at[c, pl.ds(r0, CP)])


BN = 400
GRID = N // BN


def _tc_scale_matmul_body(degp_ref, x_ref, w_ref, o_ref):
    dinv = lax.rsqrt(degp_ref[0] + degp_ref[1] + 1.0)
    o_ref[...] = dinv * jnp.dot(x_ref[...], w_ref[...],
                                preferred_element_type=jnp.float32)


def _tc_mid_body(degp_ref, p0_ref, p1_ref, g_ref, w_ref, b_ref, o_ref):
    dinv = lax.rsqrt(degp_ref[0] + degp_ref[1] + 1.0)
    h = dinv * (p0_ref[0] + p1_ref[0] + g_ref[...]) + b_ref[...]
    h = jnp.maximum(h, 0.0)
    o_ref[...] = dinv * jnp.dot(h, w_ref[...],
                                preferred_element_type=jnp.float32)


def _tc_final_body(degp_ref, p0_ref, p1_ref, g_ref, b_ref, o_ref):
    dinv = lax.rsqrt(degp_ref[0] + degp_ref[1] + 1.0)
    z = dinv * (p0_ref[0] + p1_ref[0] + g_ref[...]) + b_ref[...]
    m = jnp.max(z, axis=1, keepdims=True)
    e = jnp.exp(z - m)
    lse = jnp.log(jnp.sum(e, axis=1, keepdims=True)) + m
    o_ref[...] = z - lse


_deg_spec = pl.BlockSpec((2, BN, 1), lambda i: (0, i, 0))
_row_spec = pl.BlockSpec((BN, D), lambda i: (i, 0))
_part_spec0 = pl.BlockSpec((1, BN, D), lambda i: (0, i, 0))
_part_spec1 = pl.BlockSpec((1, BN, D), lambda i: (1, i, 0))
_w_spec = pl.BlockSpec((D, D), lambda i: (0, 0))
_b_spec = pl.BlockSpec((1, D), lambda i: (0, 0))
_out_f32 = jax.ShapeDtypeStruct((N, D), jnp.float32)


def _tc_scale_matmul(degp, x, w):
    return pl.pallas_call(
        _tc_scale_matmul_body,
        grid=(GRID,),
        in_specs=[_deg_spec, _row_spec, _w_spec],
        out_specs=_row_spec,
        out_shape=_out_f32,
    )(degp, x, w)


def _tc_mid(degp, part, g, w, b):
    return pl.pallas_call(
        _tc_mid_body,
        grid=(GRID,),
        in_specs=[_deg_spec, _part_spec0, _part_spec1, _row_spec, _w_spec,
                  _b_spec],
        out_specs=_row_spec,
        out_shape=_out_f32,
    )(degp, part, part, g, w, b)


def _tc_final(degp, part, g, b):
    return pl.pallas_call(
        _tc_final_body,
        grid=(GRID,),
        in_specs=[_deg_spec, _part_spec0, _part_spec1, _row_spec, _b_spec],
        out_specs=_row_spec,
        out_shape=_out_f32,
    )(degp, part, part, g, b)


def kernel(x, edge_index, W1, b1, W2, b2):
    x = x.astype(jnp.float32)
    src = edge_index[0].astype(jnp.int32)
    dst = edge_index[1].astype(jnp.int32)
    pad = EP - E
    srcp = jnp.concatenate([src, jnp.zeros((pad,), jnp.int32)])
    dstp = jnp.concatenate([dst, jnp.full((pad,), N, jnp.int32)])
    src3 = srcp.reshape(NW, CHW, C)
    dst3 = dstp.reshape(NW, CHW, C)
    dst_deg = dstp.reshape(NW * DCH, DC)

    ones_c = jnp.ones((DC,), jnp.float32)
    zeros_deg = jnp.zeros((DEG_PER_TILE,), jnp.float32)
    zeros_rows = jnp.zeros((CP, D), jnp.float32)
    rpad = jnp.zeros((ACC_ROWS - N, D), jnp.float32)

    degp = _sc_degree(dst_deg, ones_c, zeros_deg)
    degp = degp.reshape(NC, DEG_ACC)[:, :N].reshape(NC, N, 1)

    g1 = _tc_scale_matmul(degp, x, W1)
    g1p = jnp.concatenate([g1, rpad])
    part1, _ = _sc_scatter(g1p, src3, dst3, zeros_rows)
    g2 = _tc_mid(degp, part1, g1, W2, b1.reshape(1, D))
    g2p = jnp.concatenate([g2, rpad])
    part2, _ = _sc_scatter(g2p, src3, dst3, zeros_rows)
    return _tc_final(degp, part2, g2, b2.reshape(1, D))
